# baseline (speedup 1.0000x reference)
import jax
import jax.numpy as jnp
from jax import lax
from jax.experimental import pallas as pl
from jax.experimental.pallas import tpu as pltpu

_T = 11
_EX = 16
_SQ = 512


def _scale_kernel(num_ref, b_ref, sm_ref, hist_ref, mtl_ref, wt_ref,
                  bert_ref, nbert_ref, nmtl_ref, probs_ref):
    i = pl.program_id(0)
    bs = hist_ref.shape[1]
    w = wt_ref[0, :]
    h = hist_ref[0]
    bias = b_ref[0]
    logit = jnp.sum(h * w[None, :], axis=1) + bias
    t = lax.broadcasted_iota(jnp.int32, (bs, _T), 1)
    r = lax.broadcasted_iota(jnp.int32, (bs, _T), 0)
    lengths = sm_ref[0][:, None]
    mask = (t >= _T - lengths).astype(jnp.float32)
    rowm = (r < num_ref[0]).astype(jnp.float32)
    lrow = jnp.where(t == _T - 1, logit[:, None], bias)
    e = jnp.exp(lrow) * mask * rowm
    p = e / jnp.sum(e, axis=1, keepdims=True)
    s = p[:, _T - 1]
    probs_ref[0] = p
    nmtl_ref[0] = mtl_ref[0] * s[:, None]
    col = lax.broadcasted_iota(jnp.int32, (_EX, bs), 1)
    row = lax.broadcasted_iota(jnp.int32, (_EX, bs), 0)
    sel = (col == i * _EX + row).astype(jnp.float32)
    sblk = jnp.sum(sel * s[None, :], axis=1)
    nbert_ref[...] = bert_ref[...] * sblk[:, None, None]


def kernel(bert_representation, history_attention_input, mtl_input,
           slice_mask, slice_num, W, b):
    bs, seq, hid = bert_representation.shape
    wt = W.reshape(1, hid)
    num = jnp.asarray(slice_num, jnp.int32).reshape(1)
    grid = (bs // _EX, seq // _SQ)
    nbert, nmtl, probs = pl.pallas_call(
        _scale_kernel,
        grid=grid,
        in_specs=[
            pl.BlockSpec(memory_space=pltpu.SMEM),
            pl.BlockSpec(memory_space=pltpu.SMEM),
            pl.BlockSpec((1, bs), lambda i, j: (0, 0)),
            pl.BlockSpec((1, bs, hid), lambda i, j: (0, 0, 0)),
            pl.BlockSpec((1, bs, hid), lambda i, j: (0, 0, 0)),
            pl.BlockSpec((1, hid), lambda i, j: (0, 0)),
            pl.BlockSpec((_EX, _SQ, hid), lambda i, j: (i, j, 0)),
        ],
        out_specs=[
            pl.BlockSpec((_EX, _SQ, hid), lambda i, j: (i, j, 0)),
            pl.BlockSpec((1, bs, hid), lambda i, j: (0, 0, 0)),
            pl.BlockSpec((1, bs, _T), lambda i, j: (0, 0, 0)),
        ],
        out_shape=[
            jax.ShapeDtypeStruct((bs, seq, hid), jnp.float32),
            jax.ShapeDtypeStruct((1, bs, hid), jnp.float32),
            jax.ShapeDtypeStruct((1, bs, _T), jnp.float32),
        ],
        compiler_params=pltpu.CompilerParams(
            dimension_semantics=("arbitrary", "arbitrary"),
        ),
    )(num, b, slice_mask.astype(jnp.int32).reshape(1, bs),
      history_attention_input.reshape(1, bs, hid),
      mtl_input.reshape(1, bs, hid), wt, bert_representation)
    return nbert, nmtl.reshape(bs, hid), probs.reshape(bs, _T)


# speedup vs baseline: 1.1657x; 1.1657x over previous
import jax
import jax.numpy as jnp
from jax import lax
from jax.experimental import pallas as pl
from jax.experimental.pallas import tpu as pltpu

_T = 11
_EX = 8
_SQ = 512


def _scale_kernel(num_ref, b_ref, sm_ref, hist_ref, mtl_ref, wt_ref,
                  bert_ref, nbert_ref, nmtl_ref, probs_ref):
    i = pl.program_id(0)
    bs = hist_ref.shape[1]
    w = wt_ref[0, :]
    h = hist_ref[0]
    bias = b_ref[0]
    logit = jnp.sum(h * w[None, :], axis=1) + bias
    t = lax.broadcasted_iota(jnp.int32, (bs, _T), 1)
    r = lax.broadcasted_iota(jnp.int32, (bs, _T), 0)
    lengths = sm_ref[0][:, None]
    mask = (t >= _T - lengths).astype(jnp.float32)
    rowm = (r < num_ref[0]).astype(jnp.float32)
    lrow = jnp.where(t == _T - 1, logit[:, None], bias)
    e = jnp.exp(lrow) * mask * rowm
    p = e / jnp.sum(e, axis=1, keepdims=True)
    s = p[:, _T - 1]
    probs_ref[0] = p
    nmtl_ref[0] = mtl_ref[0] * s[:, None]
    col = lax.broadcasted_iota(jnp.int32, (_EX, bs), 1)
    row = lax.broadcasted_iota(jnp.int32, (_EX, bs), 0)
    sel = (col == i * _EX + row).astype(jnp.float32)
    sblk = jnp.sum(sel * s[None, :], axis=1)
    nbert_ref[...] = bert_ref[...] * sblk[:, None, None]


def kernel(bert_representation, history_attention_input, mtl_input,
           slice_mask, slice_num, W, b):
    bs, seq, hid = bert_representation.shape
    wt = W.reshape(1, hid)
    num = jnp.asarray(slice_num, jnp.int32).reshape(1)
    grid = (bs // _EX, seq // _SQ)
    nbert, nmtl, probs = pl.pallas_call(
        _scale_kernel,
        grid=grid,
        in_specs=[
            pl.BlockSpec(memory_space=pltpu.SMEM),
            pl.BlockSpec(memory_space=pltpu.SMEM),
            pl.BlockSpec((1, bs), lambda i, j: (0, 0)),
            pl.BlockSpec((1, bs, hid), lambda i, j: (0, 0, 0)),
            pl.BlockSpec((1, bs, hid), lambda i, j: (0, 0, 0)),
            pl.BlockSpec((1, hid), lambda i, j: (0, 0)),
            pl.BlockSpec((_EX, _SQ, hid), lambda i, j: (i, j, 0)),
        ],
        out_specs=[
            pl.BlockSpec((_EX, _SQ, hid), lambda i, j: (i, j, 0)),
            pl.BlockSpec((1, bs, hid), lambda i, j: (0, 0, 0)),
            pl.BlockSpec((1, bs, _T), lambda i, j: (0, 0, 0)),
        ],
        out_shape=[
            jax.ShapeDtypeStruct((bs, seq, hid), jnp.float32),
            jax.ShapeDtypeStruct((1, bs, hid), jnp.float32),
            jax.ShapeDtypeStruct((1, bs, _T), jnp.float32),
        ],
        compiler_params=pltpu.CompilerParams(
            dimension_semantics=("parallel", "parallel"),
        ),
    )(num, b, slice_mask.astype(jnp.int32).reshape(1, bs),
      history_attention_input.reshape(1, bs, hid),
      mtl_input.reshape(1, bs, hid), wt, bert_representation)
    return nbert, nmtl.reshape(bs, hid), probs.reshape(bs, _T)
